# baseline (speedup 1.0000x reference)
import functools

import jax
import jax.numpy as jnp
from jax import lax
from jax.experimental import pallas as pl
from jax.experimental.pallas import tpu as pltpu
from jax.experimental.pallas import tpu_sc as plsc

_LANES = 16
_NW = 32
_BB = 128

def _build_sc(vocab, maxlen, embed, batch):
  lblk = maxlen // 8
  eblk = embed // 8
  nbat = batch // _BB
  jv = _BB // _LANES

  mesh = plsc.VectorSubcoreMesh(core_axis_name="c", subcore_axis_name="s")
  nc = 2

  @functools.partial(
      pl.kernel,
      mesh=mesh,
      out_type=jax.ShapeDtypeStruct((maxlen * eblk, nbat, 8, _BB),
                                    jnp.float32),
      compiler_params=pltpu.CompilerParams(use_tc_tiling_on_sc=False,
                                           needs_layout_passes=False),
      scratch_types=[
          pltpu.VMEM((lblk, 8, _BB), jnp.int32),
          pltpu.VMEM((_BB, embed), jnp.float32),
          pltpu.VMEM((_BB, embed), jnp.float32),
          pltpu.VMEM((eblk, 8, _BB), jnp.float32),
          pltpu.VMEM((eblk, 8, _BB), jnp.float32),
          pltpu.VMEM((maxlen, embed), jnp.float32),
          pltpu.SemaphoreType.DMA,
          pltpu.SemaphoreType.DMA,
      ],
  )
  def emb(xq_hbm, tok_hbm, pos_hbm, out_hbm, idx_v, rows0, rows1, stg0, stg1,
          pos_v, gsem, osem):
    wid = lax.axis_index("s") * nc + lax.axis_index("c")
    rows = (rows0, rows1)
    stgs = (stg0, stg1)

    pltpu.sync_copy(xq_hbm.at[:, wid], idx_v)
    pltpu.sync_copy(pos_hbm, pos_v)

    def gather_desc(l, buf):
      return pltpu.make_async_copy(
          tok_hbm.at[idx_v.at[lax.div(l, 8), lax.rem(l, 8)]], buf, gsem)

    def store_desc(l, stg):
      return pltpu.make_async_copy(stg, out_hbm.at[pl.ds(l * eblk, eblk), wid],
                                   osem)

    gather_desc(0, rows[0]).start()
    iota = lax.iota(jnp.int32, _LANES)

    def pair_body(g, carry):
      for b in range(2):
        l = g * 2 + b
        buf, stg = rows[b], stgs[b]

        gather_desc(l, buf).wait()

        @pl.when(l + 1 < maxlen)
        def _():
          gather_desc(l + 1, rows[b ^ 1]).start()

        @pl.when(l >= 2)
        def _():
          store_desc(l - 2, stg).wait()

        lsplat = jnp.full((_LANES,), l, jnp.int32)

        @plsc.parallel_loop(0, (embed // _LANES) * _LANES, 1, unroll=4)
        def tr_body(t):
          q = lax.shift_right_logical(t, 4)
          d = t & (_LANES - 1)
          ecol = q * _LANES + ((iota + d) & (_LANES - 1))
          p = plsc.load_gather(pos_v, [lsplat, ecol])
          er = lax.shift_right_logical(ecol, 3)
          rr = ecol & 7
          for j in range(jv):
            rowsel = iota + (j * _LANES)
            vals = plsc.load_gather(buf, [rowsel, ecol])
            plsc.store_scatter(stg, [er, rr, rowsel], vals + p)

        store_desc(l, stg).start()
      return carry

    lax.fori_loop(0, maxlen // 2, pair_body, 0)
    store_desc(maxlen - 2, stgs[0]).wait()
    store_desc(maxlen - 1, stgs[1]).wait()

  return emb


def kernel(x, token_table, pos_table):
  batch, maxlen = x.shape
  vocab, embed = token_table.shape
  xq = (x.astype(jnp.int32).T
        .reshape(maxlen // 8, 8, batch // _BB, _BB)
        .transpose(0, 2, 1, 3))
  tok = jax.lax.optimization_barrier(
      token_table.reshape(vocab // 16, 16 * embed)).reshape(vocab, embed)
  emb = _build_sc(vocab, maxlen, embed, batch)
  out4 = emb(xq, tok, pos_table)
  out = (out4.reshape(maxlen, embed // 8, batch // _BB, 8, _BB)
         .transpose(2, 4, 0, 1, 3)
         .reshape(batch, maxlen, embed))
  return out


# speedup vs baseline: 1.3164x; 1.3164x over previous
import functools

import jax
import jax.numpy as jnp
from jax import lax
from jax.experimental import pallas as pl
from jax.experimental.pallas import tpu as pltpu
from jax.experimental.pallas import tpu_sc as plsc

_LANES = 16
_NW = 32
_BB = 128

def _build_sc(vocab, maxlen, embed, batch):
  lblk = maxlen // 8
  eblk = embed // 8
  nbat = batch // _BB
  jv = _BB // _LANES

  mesh = plsc.VectorSubcoreMesh(core_axis_name="c", subcore_axis_name="s")
  nc = 2

  @functools.partial(
      pl.kernel,
      mesh=mesh,
      out_type=jax.ShapeDtypeStruct((maxlen * eblk, nbat, 8, _BB),
                                    jnp.float32),
      compiler_params=pltpu.CompilerParams(use_tc_tiling_on_sc=False,
                                           needs_layout_passes=False),
      scratch_types=[
          pltpu.VMEM((lblk, 8, _BB), jnp.int32),
          pltpu.VMEM((_BB, embed), jnp.float32),
          pltpu.VMEM((_BB, embed), jnp.float32),
          pltpu.VMEM((eblk, 8, _BB), jnp.float32),
          pltpu.VMEM((eblk, 8, _BB), jnp.float32),
          pltpu.VMEM((maxlen, embed), jnp.float32),
          pltpu.SemaphoreType.DMA,
          pltpu.SemaphoreType.DMA,
      ],
  )
  def emb(xq_hbm, tok_hbm, pos_hbm, out_hbm, idx_v, rows0, rows1, stg0, stg1,
          pos_v, gsem, osem):
    wid = lax.axis_index("s") * nc + lax.axis_index("c")
    rows = (rows0, rows1)
    stgs = (stg0, stg1)

    pltpu.sync_copy(xq_hbm.at[:, wid], idx_v)
    pltpu.sync_copy(pos_hbm, pos_v)

    def gather_desc(l, buf):
      return pltpu.make_async_copy(
          tok_hbm.at[idx_v.at[lax.div(l, 8), lax.rem(l, 8)]], buf, gsem)

    def store_desc(l, stg):
      return pltpu.make_async_copy(stg, out_hbm.at[pl.ds(l * eblk, eblk), wid],
                                   osem)

    gather_desc(0, rows[0]).start()
    iota = lax.iota(jnp.int32, _LANES)

    def pair_body(g, carry):
      for b in range(2):
        l = g * 2 + b
        buf, stg = rows[b], stgs[b]

        gather_desc(l, buf).wait()

        @pl.when(l + 1 < maxlen)
        def _():
          gather_desc(l + 1, rows[b ^ 1]).start()

        @pl.when(l >= 2)
        def _():
          store_desc(l - 2, stg).wait()

        lsplat = jnp.full((_LANES,), l, jnp.int32)

        @plsc.parallel_loop(0, (embed // _LANES) * _LANES, 1, unroll=4)
        def tr_body(t):
          q = lax.shift_right_logical(t, 4)
          d = t & (_LANES - 1)
          ecol = q * _LANES + ((iota + d) & (_LANES - 1))
          p = plsc.load_gather(pos_v, [lsplat, ecol])
          er = lax.shift_right_logical(ecol, 3)
          rr = ecol & 7
          for j in range(jv):
            rowsel = iota + (j * _LANES)
            vals = plsc.load_gather(buf, [rowsel, ecol])
            plsc.store_scatter(stg, [er, rr, rowsel], vals + p)

        store_desc(l, stg).start()
      return carry

    lax.fori_loop(0, maxlen // 2, pair_body, 0)
    store_desc(maxlen - 2, stgs[0]).wait()
    store_desc(maxlen - 1, stgs[1]).wait()

  return emb


def kernel(x, token_table, pos_table):
  batch, maxlen = x.shape
  vocab, embed = token_table.shape
  xq = (x.astype(jnp.int32).T
        .reshape(maxlen // 8, 8, batch // _BB, _BB)
        .transpose(0, 2, 1, 3))
  tok = jax.lax.optimization_barrier(
      token_table.reshape(vocab // 2, 2 * embed)).reshape(vocab, embed)
  emb = _build_sc(vocab, maxlen, embed, batch)
  out4 = emb(xq, tok, pos_table)
  out = (out4.reshape(maxlen, embed // 8, batch // _BB, 8, _BB)
         .transpose(2, 4, 0, 1, 3)
         .reshape(batch, maxlen, embed))
  return out
